# baseline (speedup 1.0000x reference)
import functools

import jax
import jax.numpy as jnp
from jax.experimental import pallas as pl
from jax.experimental.pallas import tpu as pltpu
from jax.experimental.pallas import tpu_sc as plsc

B = 16384
D = 256
K = 8192

BM = 1024
BK = 1024
NB = B // BM
NK = K // BK

SC_CORES = 2
SC_SUBCORES = 16
NW = SC_CORES * SC_SUBCORES
BPW = B // NW
CHUNK = 256
NCHUNK = BPW // CHUNK


def _vq_tc_body(x_ref, cb2_ref, x2_ref, c2_ref, iota_ref, idx_ref, loss_ref,
                minv_ref, arg_ref):
    b = pl.program_id(0)
    k = pl.program_id(1)

    x = x_ref[...]
    cb2 = cb2_ref[...]
    mm2 = jax.lax.dot_general(
        x, cb2, (((1,), (1,)), ((), ())),
        preferred_element_type=jnp.float32)
    d2 = (x2_ref[...] + c2_ref[...]) - mm2
    m2r = jnp.min(d2, axis=1, keepdims=True)
    m2 = jnp.maximum(m2r, 0.0)
    a = jnp.sqrt(m2)

    m2w = m2.reshape(8, BM // 8)
    aw = a.reshape(8, BM // 8)
    u = (m2w + m2w * jnp.float32(2.0 ** -24)) - m2w
    p1 = m2w + u
    p2 = m2w + (u + u)
    p4 = m2w + 4.0 * u
    p8 = m2w + 8.0 * u
    t1 = jnp.sqrt(p1) == aw
    t2 = jnp.sqrt(p2) == aw
    t4 = jnp.sqrt(p4) == aw
    t8 = jnp.sqrt(p8) == aw
    tie_u = jnp.where(
        t1, jnp.where(t2, jnp.where(t4, jnp.where(t8, p8, p4), p2), p1),
        m2w).reshape(BM, 1)

    bargf = jnp.min(jnp.where(d2 <= tie_u, iota_ref[...], jnp.float32(3e8)),
                    axis=1, keepdims=True)
    barg = jnp.minimum(bargf.astype(jnp.int32) + k * BK, K - 1)
    bmin = a

    @pl.when(k == 0)
    def _():
        minv_ref[...] = bmin
        arg_ref[...] = barg

    @pl.when(k > 0)
    def _():
        pv = minv_ref[...]
        pa = arg_ref[...]
        upd = bmin < pv
        minv_ref[...] = jnp.where(upd, bmin, pv)
        arg_ref[...] = jnp.where(upd, barg, pa)

    @pl.when(k == NK - 1)
    def _():
        idx_ref[...] = arg_ref[...]
        mv = minv_ref[...]
        block_loss = jnp.sum(mv * mv)

        @pl.when(b == 0)
        def _():
            loss_ref[...] = jnp.full((1, 1), block_loss, jnp.float32)

        @pl.when(b > 0)
        def _():
            loss_ref[...] = loss_ref[...] + block_loss

        @pl.when(b == NB - 1)
        def _():
            loss_ref[...] = loss_ref[...] * (1.0 / (B * D))


_vq_tc = pl.pallas_call(
    _vq_tc_body,
    grid=(NB, NK),
    in_specs=[
        pl.BlockSpec((BM, D), lambda b, k: (b, 0)),
        pl.BlockSpec((BK, D), lambda b, k: (k, 0)),
        pl.BlockSpec((BM, 1), lambda b, k: (b, 0)),
        pl.BlockSpec((1, BK), lambda b, k: (0, k)),
        pl.BlockSpec((1, BK), lambda b, k: (0, 0)),
    ],
    out_specs=[
        pl.BlockSpec((BM, 1), lambda b, k: (b, 0)),
        pl.BlockSpec((1, 1), lambda b, k: (0, 0)),
    ],
    out_shape=[
        jax.ShapeDtypeStruct((B, 1), jnp.int32),
        jax.ShapeDtypeStruct((1, 1), jnp.float32),
    ],
    scratch_shapes=[
        pltpu.VMEM((BM, 1), jnp.float32),
        pltpu.VMEM((BM, 1), jnp.int32),
    ],
)


@functools.lru_cache(maxsize=1)
def _make_sc_gather():
    @functools.partial(
        pl.kernel,
        mesh=plsc.VectorSubcoreMesh(core_axis_name="c", subcore_axis_name="s"),
        out_type=jax.ShapeDtypeStruct((B, D), jnp.float32),
        scratch_types=[
            pltpu.VMEM((CHUNK,), jnp.int32),
            pltpu.VMEM((CHUNK, D), jnp.float32),
            pltpu.SemaphoreType.DMA,
        ],
    )
    def _sc_gather(cb_hbm, idx_hbm, out_hbm, idx_v, rows_v, sem):
        wid = jax.lax.axis_index("s") * SC_CORES + jax.lax.axis_index("c")
        base = wid * BPW
        for j in range(NCHUNK):
            off = base + j * CHUNK
            pltpu.sync_copy(idx_hbm.at[pl.ds(off, CHUNK)], idx_v)
            pltpu.async_copy(cb_hbm.at[idx_v], rows_v, sem).wait()
            pltpu.sync_copy(rows_v, out_hbm.at[pl.ds(off, CHUNK)])

    return _sc_gather


def kernel(x, codebook):
    x2 = jnp.sum(x * x, axis=1, keepdims=True)
    c2 = jnp.sum(codebook * codebook, axis=1)[None, :]
    cb2 = codebook * 2.0
    iota = jnp.arange(BK, dtype=jnp.float32)[None, :]
    idx2d, loss = _vq_tc(x, cb2, x2, c2, iota)
    indices = idx2d.reshape(B)
    quantized = _make_sc_gather()(codebook, indices)
    return quantized, indices, loss.reshape(())


# speedup vs baseline: 1.4655x; 1.4655x over previous
import functools

import jax
import jax.numpy as jnp
from jax.experimental import pallas as pl
from jax.experimental.pallas import tpu as pltpu
from jax.experimental.pallas import tpu_sc as plsc

B = 16384
D = 256
K = 8192

BM = 1024
BK = 1024
NB = B // BM
NK = K // BK

SC_CORES = 2
SC_SUBCORES = 16
NW = SC_CORES * SC_SUBCORES
BPW = B // NW
CHUNK = 256
NCHUNK = BPW // CHUNK


def _vq_tc_body(cb2_ref, x_ref, x2_ref, c2_ref, iota_ref, idx_ref, loss_ref,
                minv_ref, arg_ref):
    b = pl.program_id(0)
    k = pl.program_id(1)

    cb2 = cb2_ref[...]
    x = x_ref[...]
    mm2 = jax.lax.dot_general(
        cb2, x, (((1,), (1,)), ((), ())),
        preferred_element_type=jnp.float32)
    d2 = (x2_ref[...] + c2_ref[...]) - mm2
    m2r = jnp.min(d2, axis=0, keepdims=True)
    m2 = jnp.maximum(m2r, 0.0)
    a = jnp.sqrt(m2)

    u = (m2 + m2 * jnp.float32(2.0 ** -24)) - m2
    p1 = m2 + u
    p2 = m2 + (u + u)
    p4 = m2 + 4.0 * u
    p8 = m2 + 8.0 * u
    t1 = jnp.sqrt(p1) == a
    t2 = jnp.sqrt(p2) == a
    t4 = jnp.sqrt(p4) == a
    t8 = jnp.sqrt(p8) == a
    tie_u = jnp.where(
        t1, jnp.where(t2, jnp.where(t4, jnp.where(t8, p8, p4), p2), p1), m2)

    bargf = jnp.min(jnp.where(d2 <= tie_u, iota_ref[...], jnp.float32(3e8)),
                    axis=0, keepdims=True)
    barg = jnp.minimum(bargf.astype(jnp.int32) + k * BK, K - 1)

    @pl.when(k == 0)
    def _():
        minv_ref[...] = a
        arg_ref[...] = barg

    @pl.when(k > 0)
    def _():
        pv = minv_ref[...]
        pa = arg_ref[...]
        upd = a < pv
        minv_ref[...] = jnp.where(upd, a, pv)
        arg_ref[...] = jnp.where(upd, barg, pa)

    @pl.when(k == NK - 1)
    def _():
        idx_ref[...] = arg_ref[...]
        mv = minv_ref[...]
        block_loss = jnp.sum(mv * mv)

        @pl.when(b == 0)
        def _():
            loss_ref[...] = jnp.full((1, 1), block_loss, jnp.float32)

        @pl.when(b > 0)
        def _():
            loss_ref[...] = loss_ref[...] + block_loss

        @pl.when(b == NB - 1)
        def _():
            loss_ref[...] = loss_ref[...] * (1.0 / (B * D))


_vq_tc = pl.pallas_call(
    _vq_tc_body,
    grid=(NB, NK),
    in_specs=[
        pl.BlockSpec((BK, D), lambda b, k: (k, 0)),
        pl.BlockSpec((BM, D), lambda b, k: (b, 0)),
        pl.BlockSpec((1, BM), lambda b, k: (0, b)),
        pl.BlockSpec((BK, 1), lambda b, k: (k, 0)),
        pl.BlockSpec((BK, 1), lambda b, k: (0, 0)),
    ],
    out_specs=[
        pl.BlockSpec((1, BM), lambda b, k: (0, b)),
        pl.BlockSpec((1, 1), lambda b, k: (0, 0)),
    ],
    out_shape=[
        jax.ShapeDtypeStruct((1, B), jnp.int32),
        jax.ShapeDtypeStruct((1, 1), jnp.float32),
    ],
    scratch_shapes=[
        pltpu.VMEM((1, BM), jnp.float32),
        pltpu.VMEM((1, BM), jnp.int32),
    ],
)


@functools.lru_cache(maxsize=1)
def _make_sc_gather():
    @functools.partial(
        pl.kernel,
        mesh=plsc.VectorSubcoreMesh(core_axis_name="c", subcore_axis_name="s"),
        out_type=jax.ShapeDtypeStruct((B, D), jnp.float32),
        scratch_types=[
            pltpu.VMEM((CHUNK,), jnp.int32),
            pltpu.VMEM((CHUNK, D), jnp.float32),
            pltpu.SemaphoreType.DMA,
        ],
    )
    def _sc_gather(cb_hbm, idx_hbm, out_hbm, idx_v, rows_v, sem):
        wid = jax.lax.axis_index("s") * SC_CORES + jax.lax.axis_index("c")
        base = wid * BPW
        for j in range(NCHUNK):
            off = base + j * CHUNK
            pltpu.sync_copy(idx_hbm.at[pl.ds(off, CHUNK)], idx_v)
            pltpu.async_copy(cb_hbm.at[idx_v], rows_v, sem).wait()
            pltpu.sync_copy(rows_v, out_hbm.at[pl.ds(off, CHUNK)])

    return _sc_gather


def kernel(x, codebook):
    x2 = jnp.sum(x * x, axis=1)[None, :]
    c2 = jnp.sum(codebook * codebook, axis=1)[:, None]
    cb2 = codebook * 2.0
    iota = jnp.arange(BK, dtype=jnp.float32)[:, None]
    idx2d, loss = _vq_tc(cb2, x, x2, c2, iota)
    indices = idx2d.reshape(B)
    quantized = _make_sc_gather()(codebook, indices)
    return quantized, indices, loss.reshape(())


# speedup vs baseline: 1.7273x; 1.1787x over previous
import functools

import jax
import jax.numpy as jnp
from jax.experimental import pallas as pl
from jax.experimental.pallas import tpu as pltpu
from jax.experimental.pallas import tpu_sc as plsc

B = 16384
D = 256
K = 8192

BM = 8192
BK = 1024
NB = B // BM
NK = K // BK

SC_CORES = 2
SC_SUBCORES = 16
NW = SC_CORES * SC_SUBCORES
BPW = B // NW
CHUNK = 256
NCHUNK = BPW // CHUNK


def _vq_tc_body(cb2_ref, x_ref, x2_ref, c2_ref, iota_ref, idx_ref, loss_ref,
                minv_ref, arg_ref):
    b = pl.program_id(0)
    k = pl.program_id(1)

    cb2 = cb2_ref[...]
    x = x_ref[...]
    mm2 = jax.lax.dot_general(
        cb2, x, (((1,), (1,)), ((), ())),
        preferred_element_type=jnp.float32)
    d2 = (x2_ref[...] + c2_ref[...]) - mm2
    m2r = jnp.min(d2, axis=0, keepdims=True)
    m2 = jnp.maximum(m2r, 0.0)
    a = jnp.sqrt(m2)

    u = (m2 + m2 * jnp.float32(2.0 ** -24)) - m2
    p1 = m2 + u
    p2 = m2 + (u + u)
    p4 = m2 + 4.0 * u
    p8 = m2 + 8.0 * u
    t1 = jnp.sqrt(p1) == a
    t2 = jnp.sqrt(p2) == a
    t4 = jnp.sqrt(p4) == a
    t8 = jnp.sqrt(p8) == a
    tie_u = jnp.where(
        t1, jnp.where(t2, jnp.where(t4, jnp.where(t8, p8, p4), p2), p1), m2)

    bargf = jnp.min(jnp.where(d2 <= tie_u, iota_ref[...], jnp.float32(3e8)),
                    axis=0, keepdims=True)
    barg = jnp.minimum(bargf.astype(jnp.int32) + k * BK, K - 1)

    @pl.when(k == 0)
    def _():
        minv_ref[...] = a
        arg_ref[...] = barg

    @pl.when(k > 0)
    def _():
        pv = minv_ref[...]
        pa = arg_ref[...]
        upd = a < pv
        minv_ref[...] = jnp.where(upd, a, pv)
        arg_ref[...] = jnp.where(upd, barg, pa)

    @pl.when(k == NK - 1)
    def _():
        idx_ref[...] = arg_ref[...]
        mv = minv_ref[...]
        block_loss = jnp.sum(mv * mv)

        @pl.when(b == 0)
        def _():
            loss_ref[...] = jnp.full((1, 1), block_loss, jnp.float32)

        @pl.when(b > 0)
        def _():
            loss_ref[...] = loss_ref[...] + block_loss

        @pl.when(b == NB - 1)
        def _():
            loss_ref[...] = loss_ref[...] * (1.0 / (B * D))


_vq_tc = pl.pallas_call(
    _vq_tc_body,
    grid=(NB, NK),
    in_specs=[
        pl.BlockSpec((BK, D), lambda b, k: (k, 0)),
        pl.BlockSpec((BM, D), lambda b, k: (b, 0)),
        pl.BlockSpec((1, BM), lambda b, k: (0, b)),
        pl.BlockSpec((BK, 1), lambda b, k: (k, 0)),
        pl.BlockSpec((BK, 1), lambda b, k: (0, 0)),
    ],
    out_specs=[
        pl.BlockSpec((1, BM), lambda b, k: (0, b)),
        pl.BlockSpec((1, 1), lambda b, k: (0, 0)),
    ],
    out_shape=[
        jax.ShapeDtypeStruct((1, B), jnp.int32),
        jax.ShapeDtypeStruct((1, 1), jnp.float32),
    ],
    scratch_shapes=[
        pltpu.VMEM((1, BM), jnp.float32),
        pltpu.VMEM((1, BM), jnp.int32),
    ],
)


@functools.lru_cache(maxsize=1)
def _make_sc_gather():
    @functools.partial(
        pl.kernel,
        mesh=plsc.VectorSubcoreMesh(core_axis_name="c", subcore_axis_name="s"),
        out_type=jax.ShapeDtypeStruct((B, D), jnp.float32),
        scratch_types=[
            pltpu.VMEM((CHUNK,), jnp.int32),
            pltpu.VMEM((CHUNK, D), jnp.float32),
            pltpu.SemaphoreType.DMA,
        ],
    )
    def _sc_gather(cb_hbm, idx_hbm, out_hbm, idx_v, rows_v, sem):
        wid = jax.lax.axis_index("s") * SC_CORES + jax.lax.axis_index("c")
        base = wid * BPW
        for j in range(NCHUNK):
            off = base + j * CHUNK
            pltpu.sync_copy(idx_hbm.at[pl.ds(off, CHUNK)], idx_v)
            pltpu.async_copy(cb_hbm.at[idx_v], rows_v, sem).wait()
            pltpu.sync_copy(rows_v, out_hbm.at[pl.ds(off, CHUNK)])

    return _sc_gather


def kernel(x, codebook):
    x2 = jnp.sum(x * x, axis=1)[None, :]
    c2 = jnp.sum(codebook * codebook, axis=1)[:, None]
    cb2 = codebook * 2.0
    iota = jnp.arange(BK, dtype=jnp.float32)[:, None]
    idx2d, loss = _vq_tc(cb2, x, x2, c2, iota)
    indices = idx2d.reshape(B)
    quantized = _make_sc_gather()(codebook, indices)
    return quantized, indices, loss.reshape(())


# speedup vs baseline: 1.7483x; 1.0121x over previous
import functools

import jax
import jax.numpy as jnp
from jax.experimental import pallas as pl
from jax.experimental.pallas import tpu as pltpu
from jax.experimental.pallas import tpu_sc as plsc

B = 16384
D = 256
K = 8192

BM = 8192
BK = 1024
NB = B // BM
NK = K // BK

SC_CORES = 2
SC_SUBCORES = 16
NW = SC_CORES * SC_SUBCORES
BPW = B // NW
CHUNK = 256
NCHUNK = BPW // CHUNK


def _vq_tc_body(cb2_ref, x_ref, x2_ref, c2_ref, iota_ref, idx_ref, loss_ref,
                minv_ref, arg_ref):
    b = pl.program_id(0)
    k = pl.program_id(1)

    cb2 = cb2_ref[...] * 2.0
    x = x_ref[...]
    mm2 = jax.lax.dot_general(
        cb2, x, (((1,), (1,)), ((), ())),
        preferred_element_type=jnp.float32)
    d2 = (x2_ref[...] + c2_ref[...]) - mm2
    m2r = jnp.min(d2, axis=0, keepdims=True)
    m2 = jnp.maximum(m2r, 0.0)
    a = jnp.sqrt(m2)

    u = (m2 + m2 * jnp.float32(2.0 ** -24)) - m2
    p1 = m2 + u
    p2 = m2 + (u + u)
    p4 = m2 + 4.0 * u
    p8 = m2 + 8.0 * u
    t1 = jnp.sqrt(p1) == a
    t2 = jnp.sqrt(p2) == a
    t4 = jnp.sqrt(p4) == a
    t8 = jnp.sqrt(p8) == a
    tie_u = jnp.where(
        t1, jnp.where(t2, jnp.where(t4, jnp.where(t8, p8, p4), p2), p1), m2)

    bargf = jnp.min(jnp.where(d2 <= tie_u, iota_ref[...], jnp.float32(3e8)),
                    axis=0, keepdims=True)
    barg = jnp.minimum(bargf.astype(jnp.int32) + k * BK, K - 1)

    @pl.when(k == 0)
    def _():
        minv_ref[...] = a
        arg_ref[...] = barg

    @pl.when(k > 0)
    def _():
        pv = minv_ref[...]
        pa = arg_ref[...]
        upd = a < pv
        minv_ref[...] = jnp.where(upd, a, pv)
        arg_ref[...] = jnp.where(upd, barg, pa)

    @pl.when(k == NK - 1)
    def _():
        idx_ref[...] = arg_ref[...]
        mv = minv_ref[...]
        block_loss = jnp.sum(mv * mv)

        @pl.when(b == 0)
        def _():
            loss_ref[...] = jnp.full((1, 1), block_loss, jnp.float32)

        @pl.when(b > 0)
        def _():
            loss_ref[...] = loss_ref[...] + block_loss

        @pl.when(b == NB - 1)
        def _():
            loss_ref[...] = loss_ref[...] * (1.0 / (B * D))


_vq_tc = pl.pallas_call(
    _vq_tc_body,
    grid=(NB, NK),
    in_specs=[
        pl.BlockSpec((BK, D), lambda b, k: (k, 0)),
        pl.BlockSpec((BM, D), lambda b, k: (b, 0)),
        pl.BlockSpec((1, BM), lambda b, k: (0, b)),
        pl.BlockSpec((BK, 1), lambda b, k: (k, 0)),
        pl.BlockSpec((BK, 1), lambda b, k: (0, 0)),
    ],
    out_specs=[
        pl.BlockSpec((1, BM), lambda b, k: (0, b)),
        pl.BlockSpec((1, 1), lambda b, k: (0, 0)),
    ],
    out_shape=[
        jax.ShapeDtypeStruct((1, B), jnp.int32),
        jax.ShapeDtypeStruct((1, 1), jnp.float32),
    ],
    scratch_shapes=[
        pltpu.VMEM((1, BM), jnp.float32),
        pltpu.VMEM((1, BM), jnp.int32),
    ],
)


@functools.lru_cache(maxsize=1)
def _make_sc_gather():
    @functools.partial(
        pl.kernel,
        mesh=plsc.VectorSubcoreMesh(core_axis_name="c", subcore_axis_name="s"),
        out_type=jax.ShapeDtypeStruct((B, D), jnp.float32),
        scratch_types=[
            pltpu.VMEM((CHUNK,), jnp.int32),
            pltpu.VMEM((CHUNK, D), jnp.float32),
            pltpu.SemaphoreType.DMA,
        ],
    )
    def _sc_gather(cb_hbm, idx_hbm, out_hbm, idx_v, rows_v, sem):
        wid = jax.lax.axis_index("s") * SC_CORES + jax.lax.axis_index("c")
        base = wid * BPW
        for j in range(NCHUNK):
            off = base + j * CHUNK
            pltpu.sync_copy(idx_hbm.at[pl.ds(off, CHUNK)], idx_v)
            pltpu.async_copy(cb_hbm.at[idx_v], rows_v, sem).wait()
            pltpu.sync_copy(rows_v, out_hbm.at[pl.ds(off, CHUNK)])

    return _sc_gather


def kernel(x, codebook):
    x2 = jnp.sum(x * x, axis=1)[None, :]
    c2 = jnp.sum(codebook * codebook, axis=1)[:, None]
    iota = jnp.arange(BK, dtype=jnp.float32)[:, None]
    idx2d, loss = _vq_tc(codebook, x, x2, c2, iota)
    indices = idx2d.reshape(B)
    quantized = _make_sc_gather()(codebook, indices)
    return quantized, indices, loss.reshape(())
